# baseline (speedup 1.0000x reference)
import functools

import jax
import jax.numpy as jnp
from jax import lax
from jax.experimental import pallas as pl
from jax.experimental.pallas import tpu as pltpu
from jax.experimental.pallas import tpu_sc as plsc

NUM_CORES = 2
NUM_SUBCORES = 16
NUM_WORKERS = NUM_CORES * NUM_SUBCORES
NUM_CHUNKS = 8
NBUF = 4
LANES = 16
TC_BLOCK = 2000


def _widen_block(x_ref, o_ref):
    x = x_ref[...]
    o_ref[...] = jnp.concatenate([x, x], axis=1)


def _widen_table(table):
    vocab, dim = table.shape
    assert vocab % TC_BLOCK == 0
    return pl.pallas_call(
        _widen_block,
        grid=(vocab // TC_BLOCK,),
        in_specs=[pl.BlockSpec((TC_BLOCK, dim), lambda i: (i, 0))],
        out_specs=pl.BlockSpec((TC_BLOCK, 2 * dim), lambda i: (i, 0)),
        out_shape=jax.ShapeDtypeStruct((vocab, 2 * dim), jnp.float32),
    )(table)


def _make_gather(vocab, dim, batch):
    assert batch % (NUM_WORKERS * NUM_CHUNKS) == 0
    b_per_w = batch // NUM_WORKERS
    chunk = b_per_w // NUM_CHUNKS

    mesh = plsc.VectorSubcoreMesh(core_axis_name="c", subcore_axis_name="s")

    @functools.partial(
        pl.kernel,
        mesh=mesh,
        out_type=jax.ShapeDtypeStruct((batch, dim), jnp.float32),
        scratch_types=[
            pltpu.VMEM((b_per_w,), jnp.int32),
            pltpu.VMEM((NBUF * chunk, 2 * dim),
                       jnp.float32),
            pltpu.VMEM((b_per_w, dim), jnp.float32),
            [pltpu.SemaphoreType.DMA] * NBUF,
            [pltpu.SemaphoreType.DMA] * NUM_CHUNKS,
        ],
    )
    def gather_kernel(table_hbm, idx_hbm, out_hbm, idsv, buf, outb,
                      gsems, osems):
        wid = lax.axis_index("s") * NUM_CORES + lax.axis_index("c")
        base = wid * b_per_w
        pltpu.sync_copy(idx_hbm.at[pl.ds(base, b_per_w)], idsv)

        def fire_gather(c):
            s = c % NBUF
            return pltpu.async_copy(
                table_hbm.at[idsv.at[pl.ds(c * chunk, chunk)]],
                buf.at[pl.ds(s * chunk, chunk)],
                gsems[s],
            )

        gathers = [fire_gather(c) for c in range(NBUF)]

        writes = []
        for c in range(NUM_CHUNKS):
            s = c % NBUF
            gathers[s].wait()

            def body(r, _):
                for k in range(dim // LANES):
                    sl = pl.ds(k * LANES, LANES)
                    outb[c * chunk + r, sl] = buf[s * chunk + r, sl]
                return _

            lax.fori_loop(0, chunk, body, None)
            if c + NBUF < NUM_CHUNKS:
                gathers[s] = fire_gather(c + NBUF)
            writes.append(
                pltpu.async_copy(
                    outb.at[pl.ds(c * chunk, chunk)],
                    out_hbm.at[pl.ds(base + c * chunk, chunk)],
                    osems[c],
                )
            )
        for w in writes:
            w.wait()

    return gather_kernel


def kernel(topology_ids, embedding_table):
    vocab, dim = embedding_table.shape
    (batch,) = topology_ids.shape
    wide = _widen_table(embedding_table)
    gather = _make_gather(vocab, dim, batch)
    return gather(wide, topology_ids.astype(jnp.int32))


# speedup vs baseline: 1.5643x; 1.5643x over previous
import functools

import jax
import jax.numpy as jnp
from jax import lax
from jax.experimental import pallas as pl
from jax.experimental.pallas import tpu as pltpu
from jax.experimental.pallas import tpu_sc as plsc

NUM_CORES = 2
NUM_SUBCORES = 16
NUM_WORKERS = NUM_CORES * NUM_SUBCORES
LANES = 16
UNROLL = 4
HALF = 50048
MAIN1 = 49920


def _make_gather(dim, vocab, batch):
    rows_per_w = dim // NUM_WORKERS

    mesh = plsc.VectorSubcoreMesh(core_axis_name="c", subcore_axis_name="s")

    @functools.partial(
        pl.kernel,
        mesh=mesh,
        out_type=jax.ShapeDtypeStruct((dim, batch), jnp.float32),
        scratch_types=[
            pltpu.VMEM((batch,), jnp.int32),
            pltpu.VMEM((1, HALF), jnp.float32),
            pltpu.VMEM((1, batch), jnp.float32),
        ],
        compiler_params=pltpu.CompilerParams(needs_layout_passes=False),
    )
    def gather_kernel(table_hbm, tail_hbm, idx_hbm, out_hbm, idsv, rowv,
                      outv):
        wid = lax.axis_index("s") * NUM_CORES + lax.axis_index("c")
        pltpu.sync_copy(idx_hbm, idsv)
        zeros = jnp.zeros((LANES,), jnp.int32)
        iota = lax.iota(jnp.int32, LANES)

        def sweep(lo):
            def body(i, _):
                for j in range(UNROLL):
                    base = (i * UNROLL + j) * LANES
                    ids = idsv[pl.ds(base, LANES)] - lo
                    if lo == 0:
                        m = ids < HALF
                    else:
                        m = ids >= 0
                    x = plsc.load_gather(rowv, [zeros, ids], mask=m)
                    plsc.store_scatter(outv, [zeros, iota + base], x,
                                       mask=m)
                return _

            lax.fori_loop(0, batch // (LANES * UNROLL), body, None)

        for r in range(rows_per_w):
            d = wid * rows_per_w + r
            row = table_hbm.at[pl.ds(d, 1)]
            pltpu.sync_copy(row.at[:, pl.ds(0, HALF)], rowv)
            sweep(0)
            pltpu.sync_copy(row.at[:, pl.ds(HALF, MAIN1)],
                            rowv.at[:, pl.ds(0, MAIN1)])
            pltpu.sync_copy(tail_hbm.at[pl.ds(d, 1)],
                            rowv.at[:, pl.ds(MAIN1, 128)])
            sweep(HALF)
            pltpu.sync_copy(outv, out_hbm.at[pl.ds(d, 1)])

    return gather_kernel


def kernel(topology_ids, embedding_table):
    vocab, dim = embedding_table.shape
    (batch,) = topology_ids.shape
    table_t = embedding_table.T
    tail = jnp.pad(table_t[:, HALF + MAIN1:], ((0, 0), (0, 96)))
    gather = _make_gather(dim, vocab, batch)
    out_t = gather(table_t, tail, topology_ids.astype(jnp.int32))
    return out_t.T


# speedup vs baseline: 2.8285x; 1.8082x over previous
import functools

import jax
import jax.numpy as jnp
from jax import lax
from jax.experimental import pallas as pl
from jax.experimental.pallas import tpu as pltpu
from jax.experimental.pallas import tpu_sc as plsc

NUM_CORES = 2
NUM_SUBCORES = 16
NUM_WORKERS = NUM_CORES * NUM_SUBCORES
LANES = 16
UNROLL = 8
HALF = 50048
MAIN1 = 49920


def _make_gather(dim, vocab, batch):
    rows_per_w = dim // NUM_WORKERS

    mesh = plsc.VectorSubcoreMesh(core_axis_name="c", subcore_axis_name="s")

    @functools.partial(
        pl.kernel,
        mesh=mesh,
        out_type=jax.ShapeDtypeStruct((dim, batch), jnp.float32),
        scratch_types=[
            pltpu.VMEM((batch,), jnp.int32),
            pltpu.VMEM((1, HALF), jnp.float32),
            pltpu.VMEM((1, batch), jnp.float32),
        ],
        compiler_params=pltpu.CompilerParams(needs_layout_passes=False),
    )
    def gather_kernel(table_hbm, tail_hbm, idx_hbm, out_hbm, idsv, rowv,
                      outv):
        wid = lax.axis_index("s") * NUM_CORES + lax.axis_index("c")
        pltpu.sync_copy(idx_hbm, idsv)
        zeros = jnp.zeros((LANES,), jnp.int32)
        iota = lax.iota(jnp.int32, LANES)

        def sweep(lo):
            def body(i, _):
                bases = [(i * UNROLL + j) * LANES for j in range(UNROLL)]
                ids = [idsv[pl.ds(b, LANES)] - lo for b in bases]
                if lo == 0:
                    ms = [v < HALF for v in ids]
                else:
                    ms = [v >= 0 for v in ids]
                xs = [
                    plsc.load_gather(rowv, [zeros, v], mask=m)
                    for v, m in zip(ids, ms)
                ]
                for b, x, m in zip(bases, xs, ms):
                    plsc.store_scatter(outv, [zeros, iota + b], x, mask=m)
                return _

            lax.fori_loop(0, batch // (LANES * UNROLL), body, None)

        for r in range(rows_per_w):
            d = wid * rows_per_w + r
            row = table_hbm.at[pl.ds(d, 1)]
            pltpu.sync_copy(row.at[:, pl.ds(0, HALF)], rowv)
            sweep(0)
            pltpu.sync_copy(row.at[:, pl.ds(HALF, MAIN1)],
                            rowv.at[:, pl.ds(0, MAIN1)])
            pltpu.sync_copy(tail_hbm.at[pl.ds(d, 1)],
                            rowv.at[:, pl.ds(MAIN1, 128)])
            sweep(HALF)
            pltpu.sync_copy(outv, out_hbm.at[pl.ds(d, 1)])

    return gather_kernel


def kernel(topology_ids, embedding_table):
    vocab, dim = embedding_table.shape
    (batch,) = topology_ids.shape
    table_t = embedding_table.T
    tail = jnp.pad(table_t[:, HALF + MAIN1:], ((0, 0), (0, 96)))
    gather = _make_gather(dim, vocab, batch)
    out_t = gather(table_t, tail, topology_ids.astype(jnp.int32))
    return out_t.T


# speedup vs baseline: 2.8688x; 1.0142x over previous
import functools

import jax
import jax.numpy as jnp
from jax import lax
from jax.experimental import pallas as pl
from jax.experimental.pallas import tpu as pltpu
from jax.experimental.pallas import tpu_sc as plsc

NUM_CORES = 2
NUM_SUBCORES = 16
NUM_WORKERS = NUM_CORES * NUM_SUBCORES
LANES = 16
UNROLL = 16
HALF = 50048
MAIN1 = 49920


def _make_gather(dim, vocab, batch):
    rows_per_w = dim // NUM_WORKERS

    mesh = plsc.VectorSubcoreMesh(core_axis_name="c", subcore_axis_name="s")

    @functools.partial(
        pl.kernel,
        mesh=mesh,
        out_type=jax.ShapeDtypeStruct((dim, batch), jnp.float32),
        scratch_types=[
            pltpu.VMEM((batch,), jnp.int32),
            pltpu.VMEM((1, HALF), jnp.float32),
            pltpu.VMEM((1, batch), jnp.float32),
        ],
        compiler_params=pltpu.CompilerParams(needs_layout_passes=False),
    )
    def gather_kernel(table_hbm, tail_hbm, idx_hbm, out_hbm, idsv, rowv,
                      outv):
        wid = lax.axis_index("s") * NUM_CORES + lax.axis_index("c")
        pltpu.sync_copy(idx_hbm, idsv)
        zeros = jnp.zeros((LANES,), jnp.int32)
        iota = lax.iota(jnp.int32, LANES)

        def sweep(lo):
            def body(i, _):
                bases = [(i * UNROLL + j) * LANES for j in range(UNROLL)]
                ids = [idsv[pl.ds(b, LANES)] - lo for b in bases]
                if lo == 0:
                    ms = [v < HALF for v in ids]
                else:
                    ms = [v >= 0 for v in ids]
                xs = [
                    plsc.load_gather(rowv, [zeros, v], mask=m)
                    for v, m in zip(ids, ms)
                ]
                for b, x, m in zip(bases, xs, ms):
                    plsc.store_scatter(outv, [zeros, iota + b], x, mask=m)
                return _

            lax.fori_loop(0, batch // (LANES * UNROLL), body, None)

        for r in range(rows_per_w):
            d = wid * rows_per_w + r
            row = table_hbm.at[pl.ds(d, 1)]
            pltpu.sync_copy(row.at[:, pl.ds(0, HALF)], rowv)
            sweep(0)
            pltpu.sync_copy(row.at[:, pl.ds(HALF, MAIN1)],
                            rowv.at[:, pl.ds(0, MAIN1)])
            pltpu.sync_copy(tail_hbm.at[pl.ds(d, 1)],
                            rowv.at[:, pl.ds(MAIN1, 128)])
            sweep(HALF)
            pltpu.sync_copy(outv, out_hbm.at[pl.ds(d, 1)])

    return gather_kernel


def kernel(topology_ids, embedding_table):
    vocab, dim = embedding_table.shape
    (batch,) = topology_ids.shape
    table_t = embedding_table.T
    tail = jnp.pad(table_t[:, HALF + MAIN1:], ((0, 0), (0, 96)))
    gather = _make_gather(dim, vocab, batch)
    out_t = gather(table_t, tail, topology_ids.astype(jnp.int32))
    return out_t.T
